# baseline (speedup 1.0000x reference)
import functools

import jax
import jax.numpy as jnp
import numpy as np
from jax import lax
from jax.experimental import pallas as pl
from jax.experimental.pallas import tpu as pltpu
from jax.experimental.pallas import tpu_sc as plsc

NC, NS, L = 2, 16, 16
K = 128
KD = 80
G = 32
F32 = jnp.float32


def _deg_fn(NP, EP, GD):
    NW = NC * NS
    CHW = EP // K // NW
    NB = CHW // GD
    RPT = NP // NS
    ZR = 128
    mesh = plsc.VectorSubcoreMesh(core_axis_name="c", subcore_axis_name="s",
                                  num_cores=NC, num_subcores=NS)

    def body(dst5, ones_hbm, zeros_hbm, out, acc, idxv, onesv, zerov):
        cid = lax.axis_index("c")
        sid = lax.axis_index("s")
        w = cid * NS + sid
        pltpu.sync_copy(ones_hbm, onesv)
        pltpu.sync_copy(zeros_hbm, zerov)
        for t in range(RPT // ZR):
            pltpu.sync_copy(zerov, acc.at[pl.ds(sid * RPT + t * ZR, ZR)])
        plsc.subcore_barrier()

        def blk(g, _):
            pltpu.sync_copy(dst5.at[w, g], idxv)

            def lp(i, _):
                pltpu.sync_copy(onesv, acc.at[idxv.at[i, 0]], add=True)
                return 0
            lax.fori_loop(0, GD, lp, 0)
            return 0
        lax.fori_loop(0, NB, blk, 0)

        plsc.subcore_barrier()
        pltpu.sync_copy(acc.at[pl.ds(sid * RPT, RPT)],
                        out.at[cid, pl.ds(sid * RPT, RPT)])

    return pl.kernel(
        body,
        out_type=jax.ShapeDtypeStruct((NC, NP, L), F32),
        mesh=mesh,
        compiler_params=pltpu.CompilerParams(use_tc_tiling_on_sc=False),
        scratch_types=[
            pltpu.VMEM_SHARED((NP, L), F32),
            pltpu.VMEM((GD, 1, K), jnp.int32),
            pltpu.VMEM((K, L), F32),
            pltpu.VMEM((ZR, L), F32),
        ],
    )


KS = 64
NBUF = 4


def _pipe_blocks(table, src5, dst5, wrow, acc, srcv, dstv, rows, semg,
                 sems, NQ, NB):
    def _wait_s(b):
        pltpu.make_async_copy(rows.at[b], acc.at[dstv.at[0, 0]],
                              sems[b]).wait()

    def blk(g, _):
        pltpu.sync_copy(src5.at[wrow, g], srcv)
        pltpu.sync_copy(dst5.at[wrow, g], dstv)
        pltpu.async_copy(table.at[srcv.at[0, 0]], rows.at[0], semg[0])
        pltpu.async_copy(table.at[srcv.at[1, 0]], rows.at[1], semg[1])

        def quad(p, _):
            for b in range(NBUF):
                c = NBUF * p + b
                pltpu.make_async_copy(table.at[srcv.at[c, 0]], rows.at[b],
                                      semg[b]).wait()
                pltpu.async_copy(rows.at[b], acc.at[dstv.at[c, 0]],
                                 sems[b], add=True)
                nb = (b + 2) % NBUF
                if b < 2:
                    @pl.when(p > 0)
                    def _():
                        _wait_s(nb)
                    pltpu.async_copy(table.at[srcv.at[c + 2, 0]],
                                     rows.at[nb], semg[nb])
                else:
                    @pl.when(p < NQ - 1)
                    def _():
                        _wait_s(nb)
                        pltpu.async_copy(table.at[srcv.at[c + 2, 0]],
                                         rows.at[nb], semg[nb])
            return 0
        lax.fori_loop(0, NQ, quad, 0)
        for b in range(NBUF):
            _wait_s(b)
        return 0
    lax.fori_loop(0, NB, blk, 0)


def _zero_acc(rows, acc, sid, RPT, ZR, D2):
    zero_row = jnp.zeros((L,), F32)

    def fill(i, _):
        for j in range(D2 // L):
            rows[0, i, pl.ds(j * L, L)] = zero_row
        return 0
    lax.fori_loop(0, ZR, fill, 0)
    for t in range(RPT // ZR):
        pltpu.sync_copy(rows.at[0], acc.at[pl.ds(sid * RPT + t * ZR, ZR)])


def _scatter_edges_fn(NP, EP, D, G2):
    NW = NC * NS
    CHW = EP // KS // NW
    NB = CHW // G2
    NQ = G2 // NBUF
    RPT = NP // NS
    ZR = KS
    mesh = plsc.VectorSubcoreMesh(core_axis_name="c", subcore_axis_name="s",
                                  num_cores=NC, num_subcores=NS)

    def body(table, src5, dst5, out, acc, srcv, dstv, rows, *sems):
        cid = lax.axis_index("c")
        sid = lax.axis_index("s")
        w = cid * NS + sid
        _zero_acc(rows, acc, sid, RPT, ZR, D)
        plsc.subcore_barrier()
        semg, sems_ = sems[:NBUF], sems[NBUF:]
        _pipe_blocks(table, src5, dst5, w, acc, srcv, dstv, rows,
                     semg, sems_, NQ, NB)
        plsc.subcore_barrier()
        pltpu.sync_copy(acc.at[pl.ds(sid * RPT, RPT)],
                        out.at[cid, pl.ds(sid * RPT, RPT)])

    return pl.kernel(
        body,
        out_type=jax.ShapeDtypeStruct((NC, NP, D), F32),
        mesh=mesh,
        compiler_params=pltpu.CompilerParams(use_tc_tiling_on_sc=False),
        scratch_types=[
            pltpu.VMEM_SHARED((NP, D), F32),
            pltpu.VMEM((G2, 1, KS), jnp.int32),
            pltpu.VMEM((G2, 1, KS), jnp.int32),
            pltpu.VMEM((NBUF, KS, D), F32),
        ] + [pltpu.SemaphoreType.DMA] * (2 * NBUF),
    )


BR = 2048


def _row_spec(D):
    return pl.BlockSpec((BR, D), lambda i: (i, 0))


def _full_spec(shape):
    nd = len(shape)
    return pl.BlockSpec(shape, lambda i: (0,) * nd)


def _tc_pre(x, degA, degB):
    NP = x.shape[0]
    din = x.shape[1]

    def body(x_ref, da_ref, db_ref, t_ref, dinv_ref):
        dsum = da_ref[...] + db_ref[...]
        deg = 1.0 + dsum[:, 0:1]
        dinv = lax.rsqrt(deg)
        t_ref[...] = x_ref[...] * dinv
        dinv_ref[...] = dinv

    return pl.pallas_call(
        body,
        grid=(NP // BR,),
        in_specs=[_row_spec(din), _row_spec(L), _row_spec(L)],
        out_specs=[_row_spec(din), _row_spec(1)],
        out_shape=[
            jax.ShapeDtypeStruct((NP, din), F32),
            jax.ShapeDtypeStruct((NP, 1), F32),
        ],
    )(x, degA, degB)


def _tc_mid(S1a, S1b, t1, dinv, b1, W1, W2, n_valid):
    NP = S1a.shape[0]
    din = t1.shape[1]
    dout = W2.shape[1]

    def body(sa_ref, sb_ref, t_ref, dinv_ref, b1_ref, w1_ref, w2_ref,
             o_ref):
        dinv = dinv_ref[...]
        u = sa_ref[...] + sb_ref[...] + t_ref[...]
        h1 = jnp.dot(u, w1_ref[...], preferred_element_type=F32)
        z = jnp.maximum(dinv * h1 + b1_ref[...], 0.0)
        h2 = jnp.dot(z, w2_ref[...], preferred_element_type=F32)
        row = (pl.program_id(0) * BR
               + lax.broadcasted_iota(jnp.int32, (BR, 1), 0))
        o_ref[...] = jnp.where(row < n_valid, h2 * dinv, 0.0)

    return pl.pallas_call(
        body,
        grid=(NP // BR,),
        in_specs=[_row_spec(din), _row_spec(din), _row_spec(din),
                  _row_spec(1), _full_spec(b1.shape),
                  _full_spec(W1.shape), _full_spec(W2.shape)],
        out_specs=_row_spec(dout),
        out_shape=jax.ShapeDtypeStruct((NP, dout), F32),
    )(S1a, S1b, t1, dinv, b1, W1, W2)


def _tc_post(S2a, S2b, h2p, dinv, b2, n_valid):
    D = S2a.shape[1]
    BP = 2000
    assert n_valid % BP == 0

    def _spec(d):
        return pl.BlockSpec((BP, d), lambda i: (i, 0))

    def body(sa_ref, sb_ref, h_ref, dinv_ref, b2_ref, out_ref):
        dinv = dinv_ref[...]
        out_ref[...] = dinv * (sa_ref[...] + sb_ref[...] + h_ref[...]) \
            + b2_ref[...]

    return pl.pallas_call(
        body,
        grid=(n_valid // BP,),
        in_specs=[_spec(D), _spec(D), _spec(D),
                  _spec(1), _full_spec(b2.shape)],
        out_specs=_spec(D),
        out_shape=jax.ShapeDtypeStruct((n_valid, D), F32),
    )(S2a, S2b, h2p, dinv, b2)


def kernel(x, edge_index, W1, b1, W2, b2):
    N, din = x.shape
    E = edge_index.shape[1]
    dh = W1.shape[1]
    dout = W2.shape[1]
    NP = ((N + BR - 1) // BR) * BR

    xp = jnp.pad(x, ((0, NP - N), (0, 0)))

    G2, GD = 32, 16
    blk_edges = NC * NS * G2 * KS
    EP = ((E + blk_edges - 1) // blk_edges) * blk_edges
    pad = EP - E
    pad_src = jnp.asarray(
        N + (np.arange(pad, dtype=np.int32) % (NP - N)))
    pad_dst = jnp.asarray((np.arange(pad, dtype=np.int32) * 61) % N)
    src_p = jnp.concatenate([edge_index[0], pad_src])
    dst_p = jnp.concatenate([edge_index[1], pad_dst])
    srcE = src_p.reshape(NC * NS, EP // KS // (NC * NS) // G2, G2, 1, KS)
    dstE = dst_p.reshape(NC * NS, EP // KS // (NC * NS) // G2, G2, 1, KS)

    dst_deg = jnp.concatenate([edge_index[1], pad_src])
    dstD = dst_deg.reshape(NC * NS, EP // K // (NC * NS) // GD, GD, 1, K)
    ones16 = jnp.asarray(np.eye(1, L, dtype=np.float32)[0]
                         * np.ones((K, 1), np.float32))
    zeros16 = jnp.zeros((128, L), F32)

    scat = _scatter_edges_fn(NP, EP, din, G2)
    degpart = _deg_fn(NP, EP, GD)(dstD, ones16, zeros16)
    t1, dinv = _tc_pre(xp, degpart[0], degpart[1])
    S1 = scat(t1, srcE, dstE)
    h2p = _tc_mid(S1[0], S1[1], t1, dinv, b1.reshape(1, -1), W1, W2, N)
    S2 = scat(h2p, srcE, dstE)
    return _tc_post(S2[0], S2[1], h2p, dinv, b2.reshape(1, -1), N)


# speedup vs baseline: 1.0072x; 1.0072x over previous
import functools

import jax
import jax.numpy as jnp
import numpy as np
from jax import lax
from jax.experimental import pallas as pl
from jax.experimental.pallas import tpu as pltpu
from jax.experimental.pallas import tpu_sc as plsc

NC, NS, L = 2, 16, 16
K = 128
KD = 80
G = 32
F32 = jnp.float32


def _deg_fn(NP, EP, GD):
    NW = NC * NS
    CHW = EP // K // NW
    NB = CHW // GD
    RPT = NP // NS
    ZR = 128
    mesh = plsc.VectorSubcoreMesh(core_axis_name="c", subcore_axis_name="s",
                                  num_cores=NC, num_subcores=NS)

    def body(dst5, ones_hbm, zeros_hbm, out, acc, idxv, onesv, zerov):
        cid = lax.axis_index("c")
        sid = lax.axis_index("s")
        w = cid * NS + sid
        pltpu.sync_copy(ones_hbm, onesv)
        pltpu.sync_copy(zeros_hbm, zerov)
        for t in range(RPT // ZR):
            pltpu.sync_copy(zerov, acc.at[pl.ds(sid * RPT + t * ZR, ZR)])
        plsc.subcore_barrier()

        def blk(g, _):
            pltpu.sync_copy(dst5.at[w, g], idxv)

            def lp(i, _):
                pltpu.sync_copy(onesv, acc.at[idxv.at[i, 0]], add=True)
                return 0
            lax.fori_loop(0, GD, lp, 0)
            return 0
        lax.fori_loop(0, NB, blk, 0)

        plsc.subcore_barrier()
        pltpu.sync_copy(acc.at[pl.ds(sid * RPT, RPT)],
                        out.at[cid, pl.ds(sid * RPT, RPT)])

    return pl.kernel(
        body,
        out_type=jax.ShapeDtypeStruct((NC, NP, L), F32),
        mesh=mesh,
        compiler_params=pltpu.CompilerParams(use_tc_tiling_on_sc=False),
        scratch_types=[
            pltpu.VMEM_SHARED((NP, L), F32),
            pltpu.VMEM((GD, 1, K), jnp.int32),
            pltpu.VMEM((K, L), F32),
            pltpu.VMEM((ZR, L), F32),
        ],
    )


KS = 64
NBUF = 4


def _pipe_blocks(table, src5, dst5, wrow, acc, srcv, dstv, rows, semg,
                 sems, NQ, NB):
    def _wait_s(b):
        pltpu.make_async_copy(rows.at[b], acc.at[dstv.at[0, 0]],
                              sems[b]).wait()

    def blk(g, _):
        pltpu.sync_copy(src5.at[wrow, g], srcv)
        pltpu.sync_copy(dst5.at[wrow, g], dstv)
        pltpu.async_copy(table.at[srcv.at[0, 0]], rows.at[0], semg[0])
        pltpu.async_copy(table.at[srcv.at[1, 0]], rows.at[1], semg[1])

        def quad(p, _):
            for b in range(NBUF):
                c = NBUF * p + b
                pltpu.make_async_copy(table.at[srcv.at[c, 0]], rows.at[b],
                                      semg[b]).wait()
                pltpu.async_copy(rows.at[b], acc.at[dstv.at[c, 0]],
                                 sems[b], add=True)
                nb = (b + 2) % NBUF
                if b < 2:
                    @pl.when(p > 0)
                    def _():
                        _wait_s(nb)
                    pltpu.async_copy(table.at[srcv.at[c + 2, 0]],
                                     rows.at[nb], semg[nb])
                else:
                    @pl.when(p < NQ - 1)
                    def _():
                        _wait_s(nb)
                        pltpu.async_copy(table.at[srcv.at[c + 2, 0]],
                                         rows.at[nb], semg[nb])
            return 0
        lax.fori_loop(0, NQ, quad, 0)
        for b in range(NBUF):
            _wait_s(b)
        return 0
    lax.fori_loop(0, NB, blk, 0)


def _zero_acc(rows, acc, sid, RPT, ZR, D2):
    zero_row = jnp.zeros((L,), F32)

    def fill(i, _):
        for j in range(D2 // L):
            rows[0, i, pl.ds(j * L, L)] = zero_row
        return 0
    lax.fori_loop(0, ZR, fill, 0)
    for t in range(RPT // ZR):
        pltpu.sync_copy(rows.at[0], acc.at[pl.ds(sid * RPT + t * ZR, ZR)])


def _scatter_edges_fn(NP, EP, D, G2):
    NW = NC * NS
    CHW = EP // KS // NW
    NB = CHW // G2
    NQ = G2 // NBUF
    RPT = NP // NS
    ZR = KS
    mesh = plsc.VectorSubcoreMesh(core_axis_name="c", subcore_axis_name="s",
                                  num_cores=NC, num_subcores=NS)

    def body(table, src5, dst5, out, acc, srcv, dstv, rows, *sems):
        cid = lax.axis_index("c")
        sid = lax.axis_index("s")
        w = cid * NS + sid
        _zero_acc(rows, acc, sid, RPT, ZR, D)
        plsc.subcore_barrier()
        semg, sems_ = sems[:NBUF], sems[NBUF:]
        _pipe_blocks(table, src5, dst5, w, acc, srcv, dstv, rows,
                     semg, sems_, NQ, NB)
        plsc.subcore_barrier()
        pltpu.sync_copy(acc.at[pl.ds(sid * RPT, RPT)],
                        out.at[cid, pl.ds(sid * RPT, RPT)])

    return pl.kernel(
        body,
        out_type=jax.ShapeDtypeStruct((NC, NP, D), F32),
        mesh=mesh,
        compiler_params=pltpu.CompilerParams(use_tc_tiling_on_sc=False),
        scratch_types=[
            pltpu.VMEM_SHARED((NP, D), F32),
            pltpu.VMEM((G2, 1, KS), jnp.int32),
            pltpu.VMEM((G2, 1, KS), jnp.int32),
            pltpu.VMEM((NBUF, KS, D), F32),
        ] + [pltpu.SemaphoreType.DMA] * (2 * NBUF),
    )


BR = 2048


def _row_spec(D):
    return pl.BlockSpec((BR, D), lambda i: (i, 0))


def _full_spec(shape):
    nd = len(shape)
    return pl.BlockSpec(shape, lambda i: (0,) * nd)


def _tc_pre(x, degA, degB, NP):
    N, din = x.shape

    def body(x_ref, da_ref, db_ref, t_ref, dinv_ref):
        dsum = da_ref[...] + db_ref[...]
        deg = 1.0 + dsum[:, 0:1]
        dinv = lax.rsqrt(deg)
        row = (pl.program_id(0) * BR
               + lax.broadcasted_iota(jnp.int32, (BR, 1), 0))
        t_ref[...] = jnp.where(row < N, x_ref[...] * dinv, 0.0)
        dinv_ref[...] = dinv

    return pl.pallas_call(
        body,
        grid=(NP // BR,),
        in_specs=[_row_spec(din), _row_spec(L), _row_spec(L)],
        out_specs=[_row_spec(din), _row_spec(1)],
        out_shape=[
            jax.ShapeDtypeStruct((NP, din), F32),
            jax.ShapeDtypeStruct((NP, 1), F32),
        ],
    )(x, degA, degB)


def _tc_mid(S1a, S1b, t1, dinv, b1, W1, W2, n_valid):
    NP = S1a.shape[0]
    din = t1.shape[1]
    dout = W2.shape[1]

    def body(sa_ref, sb_ref, t_ref, dinv_ref, b1_ref, w1_ref, w2_ref,
             o_ref):
        dinv = dinv_ref[...]
        u = sa_ref[...] + sb_ref[...] + t_ref[...]
        h1 = jnp.dot(u, w1_ref[...], preferred_element_type=F32)
        z = jnp.maximum(dinv * h1 + b1_ref[...], 0.0)
        h2 = jnp.dot(z, w2_ref[...], preferred_element_type=F32)
        row = (pl.program_id(0) * BR
               + lax.broadcasted_iota(jnp.int32, (BR, 1), 0))
        o_ref[...] = jnp.where(row < n_valid, h2 * dinv, 0.0)

    return pl.pallas_call(
        body,
        grid=(NP // BR,),
        in_specs=[_row_spec(din), _row_spec(din), _row_spec(din),
                  _row_spec(1), _full_spec(b1.shape),
                  _full_spec(W1.shape), _full_spec(W2.shape)],
        out_specs=_row_spec(dout),
        out_shape=jax.ShapeDtypeStruct((NP, dout), F32),
    )(S1a, S1b, t1, dinv, b1, W1, W2)


def _tc_post(S2a, S2b, h2p, dinv, b2, n_valid):
    D = S2a.shape[1]
    BP = 2000
    assert n_valid % BP == 0

    def _spec(d):
        return pl.BlockSpec((BP, d), lambda i: (i, 0))

    def body(sa_ref, sb_ref, h_ref, dinv_ref, b2_ref, out_ref):
        dinv = dinv_ref[...]
        out_ref[...] = dinv * (sa_ref[...] + sb_ref[...] + h_ref[...]) \
            + b2_ref[...]

    return pl.pallas_call(
        body,
        grid=(n_valid // BP,),
        in_specs=[_spec(D), _spec(D), _spec(D),
                  _spec(1), _full_spec(b2.shape)],
        out_specs=_spec(D),
        out_shape=jax.ShapeDtypeStruct((n_valid, D), F32),
    )(S2a, S2b, h2p, dinv, b2)


def kernel(x, edge_index, W1, b1, W2, b2):
    N, din = x.shape
    E = edge_index.shape[1]
    dh = W1.shape[1]
    dout = W2.shape[1]
    NP = ((N + BR - 1) // BR) * BR

    G2, GD = 32, 16
    blk_edges = NC * NS * G2 * KS
    EP = ((E + blk_edges - 1) // blk_edges) * blk_edges
    pad = EP - E
    pad_src = jnp.asarray(
        N + (np.arange(pad, dtype=np.int32) % (NP - N)))
    pad_dst = jnp.asarray((np.arange(pad, dtype=np.int32) * 61) % N)
    src_p = jnp.concatenate([edge_index[0], pad_src])
    dst_p = jnp.concatenate([edge_index[1], pad_dst])
    srcE = src_p.reshape(NC * NS, EP // KS // (NC * NS) // G2, G2, 1, KS)
    dstE = dst_p.reshape(NC * NS, EP // KS // (NC * NS) // G2, G2, 1, KS)

    dst_deg = jnp.concatenate([edge_index[1], pad_src])
    dstD = dst_deg.reshape(NC * NS, EP // K // (NC * NS) // GD, GD, 1, K)
    ones16 = jnp.asarray(np.eye(1, L, dtype=np.float32)[0]
                         * np.ones((K, 1), np.float32))
    zeros16 = jnp.zeros((128, L), F32)

    scat = _scatter_edges_fn(NP, EP, din, G2)
    degpart = _deg_fn(NP, EP, GD)(dstD, ones16, zeros16)
    t1, dinv = _tc_pre(x, degpart[0], degpart[1], NP)
    S1 = scat(t1, srcE, dstE)
    h2p = _tc_mid(S1[0], S1[1], t1, dinv, b1.reshape(1, -1), W1, W2, N)
    S2 = scat(h2p, srcE, dstE)
    return _tc_post(S2[0], S2[1], h2p, dinv, b2.reshape(1, -1), N)


# speedup vs baseline: 1.0199x; 1.0127x over previous
import functools

import jax
import jax.numpy as jnp
import numpy as np
from jax import lax
from jax.experimental import pallas as pl
from jax.experimental.pallas import tpu as pltpu
from jax.experimental.pallas import tpu_sc as plsc

NC, NS, L = 2, 16, 16
K = 128
KD = 80
G = 32
F32 = jnp.float32


def _deg_fn(NP, EP, GD):
    NW = NC * NS
    CHW = EP // K // NW
    NB = CHW // GD
    RPT = NP // NS
    ZR = 128
    mesh = plsc.VectorSubcoreMesh(core_axis_name="c", subcore_axis_name="s",
                                  num_cores=NC, num_subcores=NS)

    def body(dst5, ones_hbm, zeros_hbm, out, acc, idxv, onesv, zerov,
             sem0, sem1):
        cid = lax.axis_index("c")
        sid = lax.axis_index("s")
        w = cid * NS + sid
        sems = (sem0, sem1)
        pltpu.sync_copy(ones_hbm, onesv)
        pltpu.sync_copy(zeros_hbm, zerov)
        for t in range(RPT // ZR):
            pltpu.sync_copy(zerov, acc.at[pl.ds(sid * RPT + t * ZR, ZR)])
        plsc.subcore_barrier()

        def _wait_adds(p):
            for i in range(GD):
                pltpu.make_async_copy(onesv, acc.at[idxv.at[p, 0, 0]],
                                      sems[p]).wait()

        pltpu.sync_copy(dst5.at[w, 0], idxv.at[0])
        for g in range(NB):
            p = g % 2
            for i in range(GD):
                pltpu.async_copy(onesv, acc.at[idxv.at[p, i, 0]],
                                 sems[p], add=True)
            if g < NB - 1:
                if g > 0:
                    _wait_adds(1 - p)
                pltpu.sync_copy(dst5.at[w, g + 1], idxv.at[1 - p])
        _wait_adds((NB - 2) % 2)
        _wait_adds((NB - 1) % 2)

        plsc.subcore_barrier()
        pltpu.sync_copy(acc.at[pl.ds(sid * RPT, RPT)],
                        out.at[cid, pl.ds(sid * RPT, RPT)])

    return pl.kernel(
        body,
        out_type=jax.ShapeDtypeStruct((NC, NP, L), F32),
        mesh=mesh,
        compiler_params=pltpu.CompilerParams(use_tc_tiling_on_sc=False),
        scratch_types=[
            pltpu.VMEM_SHARED((NP, L), F32),
            pltpu.VMEM((2, GD, 1, K), jnp.int32),
            pltpu.VMEM((K, L), F32),
            pltpu.VMEM((ZR, L), F32),
        ] + [pltpu.SemaphoreType.DMA] * 2,
    )


KS = 64
NBUF = 4


def _pipe_blocks(table, src5, dst5, wrow, acc, srcv, dstv, rows, semg,
                 sems, NQ, NB):
    def _wait_s(b):
        pltpu.make_async_copy(rows.at[b], acc.at[dstv.at[0, 0]],
                              sems[b]).wait()

    def blk(g, _):
        pltpu.sync_copy(src5.at[wrow, g], srcv)
        pltpu.sync_copy(dst5.at[wrow, g], dstv)
        pltpu.async_copy(table.at[srcv.at[0, 0]], rows.at[0], semg[0])
        pltpu.async_copy(table.at[srcv.at[1, 0]], rows.at[1], semg[1])

        def quad(p, _):
            for b in range(NBUF):
                c = NBUF * p + b
                pltpu.make_async_copy(table.at[srcv.at[c, 0]], rows.at[b],
                                      semg[b]).wait()
                pltpu.async_copy(rows.at[b], acc.at[dstv.at[c, 0]],
                                 sems[b], add=True)
                nb = (b + 2) % NBUF
                if b < 2:
                    @pl.when(p > 0)
                    def _():
                        _wait_s(nb)
                    pltpu.async_copy(table.at[srcv.at[c + 2, 0]],
                                     rows.at[nb], semg[nb])
                else:
                    @pl.when(p < NQ - 1)
                    def _():
                        _wait_s(nb)
                        pltpu.async_copy(table.at[srcv.at[c + 2, 0]],
                                         rows.at[nb], semg[nb])
            return 0
        lax.fori_loop(0, NQ, quad, 0)
        for b in range(NBUF):
            _wait_s(b)
        return 0
    lax.fori_loop(0, NB, blk, 0)


def _zero_acc(rows, acc, sid, RPT, ZR, D2):
    zero_row = jnp.zeros((L,), F32)

    def fill(i, _):
        for j in range(D2 // L):
            rows[0, i, pl.ds(j * L, L)] = zero_row
        return 0
    lax.fori_loop(0, ZR, fill, 0)
    for t in range(RPT // ZR):
        pltpu.sync_copy(rows.at[0], acc.at[pl.ds(sid * RPT + t * ZR, ZR)])


def _scatter_edges_fn(NP, EP, D, G2):
    NW = NC * NS
    CHW = EP // KS // NW
    NB = CHW // G2
    NQ = G2 // NBUF
    RPT = NP // NS
    ZR = KS
    mesh = plsc.VectorSubcoreMesh(core_axis_name="c", subcore_axis_name="s",
                                  num_cores=NC, num_subcores=NS)

    def body(table, src5, dst5, out, acc, srcv, dstv, rows, *sems):
        cid = lax.axis_index("c")
        sid = lax.axis_index("s")
        w = cid * NS + sid
        _zero_acc(rows, acc, sid, RPT, ZR, D)
        plsc.subcore_barrier()
        semg, sems_ = sems[:NBUF], sems[NBUF:]
        _pipe_blocks(table, src5, dst5, w, acc, srcv, dstv, rows,
                     semg, sems_, NQ, NB)
        plsc.subcore_barrier()
        pltpu.sync_copy(acc.at[pl.ds(sid * RPT, RPT)],
                        out.at[cid, pl.ds(sid * RPT, RPT)])

    return pl.kernel(
        body,
        out_type=jax.ShapeDtypeStruct((NC, NP, D), F32),
        mesh=mesh,
        compiler_params=pltpu.CompilerParams(use_tc_tiling_on_sc=False),
        scratch_types=[
            pltpu.VMEM_SHARED((NP, D), F32),
            pltpu.VMEM((G2, 1, KS), jnp.int32),
            pltpu.VMEM((G2, 1, KS), jnp.int32),
            pltpu.VMEM((NBUF, KS, D), F32),
        ] + [pltpu.SemaphoreType.DMA] * (2 * NBUF),
    )


BR = 2048


def _row_spec(D):
    return pl.BlockSpec((BR, D), lambda i: (i, 0))


def _full_spec(shape):
    nd = len(shape)
    return pl.BlockSpec(shape, lambda i: (0,) * nd)


def _tc_pre(x, degA, degB, NP):
    N, din = x.shape

    def body(x_ref, da_ref, db_ref, t_ref, dinv_ref):
        dsum = da_ref[...] + db_ref[...]
        deg = 1.0 + dsum[:, 0:1]
        dinv = lax.rsqrt(deg)
        row = (pl.program_id(0) * BR
               + lax.broadcasted_iota(jnp.int32, (BR, 1), 0))
        t_ref[...] = jnp.where(row < N, x_ref[...] * dinv, 0.0)
        dinv_ref[...] = dinv

    return pl.pallas_call(
        body,
        grid=(NP // BR,),
        in_specs=[_row_spec(din), _row_spec(L), _row_spec(L)],
        out_specs=[_row_spec(din), _row_spec(1)],
        out_shape=[
            jax.ShapeDtypeStruct((NP, din), F32),
            jax.ShapeDtypeStruct((NP, 1), F32),
        ],
    )(x, degA, degB)


def _tc_mid(S1a, S1b, t1, dinv, b1, W1, W2, n_valid):
    NP = S1a.shape[0]
    din = t1.shape[1]
    dout = W2.shape[1]

    def body(sa_ref, sb_ref, t_ref, dinv_ref, b1_ref, w1_ref, w2_ref,
             o_ref):
        dinv = dinv_ref[...]
        u = sa_ref[...] + sb_ref[...] + t_ref[...]
        h1 = jnp.dot(u, w1_ref[...], preferred_element_type=F32)
        z = jnp.maximum(dinv * h1 + b1_ref[...], 0.0)
        h2 = jnp.dot(z, w2_ref[...], preferred_element_type=F32)
        row = (pl.program_id(0) * BR
               + lax.broadcasted_iota(jnp.int32, (BR, 1), 0))
        o_ref[...] = jnp.where(row < n_valid, h2 * dinv, 0.0)

    return pl.pallas_call(
        body,
        grid=(NP // BR,),
        in_specs=[_row_spec(din), _row_spec(din), _row_spec(din),
                  _row_spec(1), _full_spec(b1.shape),
                  _full_spec(W1.shape), _full_spec(W2.shape)],
        out_specs=_row_spec(dout),
        out_shape=jax.ShapeDtypeStruct((NP, dout), F32),
    )(S1a, S1b, t1, dinv, b1, W1, W2)


def _tc_post(S2a, S2b, h2p, dinv, b2, n_valid):
    D = S2a.shape[1]
    BP = 2000
    assert n_valid % BP == 0

    def _spec(d):
        return pl.BlockSpec((BP, d), lambda i: (i, 0))

    def body(sa_ref, sb_ref, h_ref, dinv_ref, b2_ref, out_ref):
        dinv = dinv_ref[...]
        out_ref[...] = dinv * (sa_ref[...] + sb_ref[...] + h_ref[...]) \
            + b2_ref[...]

    return pl.pallas_call(
        body,
        grid=(n_valid // BP,),
        in_specs=[_spec(D), _spec(D), _spec(D),
                  _spec(1), _full_spec(b2.shape)],
        out_specs=_spec(D),
        out_shape=jax.ShapeDtypeStruct((n_valid, D), F32),
    )(S2a, S2b, h2p, dinv, b2)


def kernel(x, edge_index, W1, b1, W2, b2):
    N, din = x.shape
    E = edge_index.shape[1]
    dh = W1.shape[1]
    dout = W2.shape[1]
    NP = ((N + BR - 1) // BR) * BR

    G2, GD = 32, 16
    blk_edges = NC * NS * G2 * KS
    EP = ((E + blk_edges - 1) // blk_edges) * blk_edges
    pad = EP - E
    pad_src = jnp.asarray(
        N + (np.arange(pad, dtype=np.int32) % (NP - N)))
    pad_dst = jnp.asarray((np.arange(pad, dtype=np.int32) * 61) % N)
    src_p = jnp.concatenate([edge_index[0], pad_src])
    dst_p = jnp.concatenate([edge_index[1], pad_dst])
    srcE = src_p.reshape(NC * NS, EP // KS // (NC * NS) // G2, G2, 1, KS)
    dstE = dst_p.reshape(NC * NS, EP // KS // (NC * NS) // G2, G2, 1, KS)

    dst_deg = jnp.concatenate([edge_index[1], pad_src])
    dstD = dst_deg.reshape(NC * NS, EP // K // (NC * NS) // GD, GD, 1, K)
    ones16 = jnp.asarray(np.eye(1, L, dtype=np.float32)[0]
                         * np.ones((K, 1), np.float32))
    zeros16 = jnp.zeros((128, L), F32)

    scat = _scatter_edges_fn(NP, EP, din, G2)
    degpart = _deg_fn(NP, EP, GD)(dstD, ones16, zeros16)
    t1, dinv = _tc_pre(x, degpart[0], degpart[1], NP)
    S1 = scat(t1, srcE, dstE)
    h2p = _tc_mid(S1[0], S1[1], t1, dinv, b1.reshape(1, -1), W1, W2, N)
    S2 = scat(h2p, srcE, dstE)
    return _tc_post(S2[0], S2[1], h2p, dinv, b2.reshape(1, -1), N)
